# baseline (speedup 1.0000x reference)
import numpy as np
import jax
import jax.numpy as jnp
from jax.experimental import pallas as pl

D = 768
H = 12
DH = 64
E = 8
FF = 3072
KD = 256
KS = 10
S = 2048
DEPTH = 1.25
SCALE = 1.0 / np.sqrt(DH)
BS = 256
BA = 128
NPROJ = 9 * D + 128
NB = 640
NEG = -1e30


def _f32(*dots):
    return dict(preferred_element_type=jnp.float32)


def _gates_kernel(x_ref, aw1_ref, ab1_ref, aw2_ref, ab2_ref,
                  tw_ref, tb_ref, gw_ref, gb_ref, o_ref):
    avg = jnp.mean(x_ref[...], axis=0, keepdims=True)
    h = jax.nn.gelu(jnp.dot(avg, aw1_ref[...],
                            preferred_element_type=jnp.float32) + ab1_ref[...])
    gwv = jax.nn.sigmoid(jnp.dot(h, aw2_ref[...],
                                 preferred_element_type=jnp.float32) + ab2_ref[...])
    tg = jax.nn.softmax(jnp.dot(avg, tw_ref[...],
                                preferred_element_type=jnp.float32) + tb_ref[...], -1)
    ag = jax.nn.softmax(jnp.dot(avg, gw_ref[...],
                                preferred_element_type=jnp.float32) + gb_ref[...], -1)
    vals = jnp.concatenate(
        [gwv, tg, ag, jnp.zeros((1, 122), jnp.float32)], axis=1)
    o_ref[...] = vals


def _proj_kernel(x_ref, w_ref, b_ref, o_ref):
    o_ref[...] = jnp.dot(x_ref[...], w_ref[...],
                         preferred_element_type=jnp.float32) + b_ref[...]


def _attn_kernel(q_ref, k_ref, v_ref, o_ref):
    for h in range(H):
        sl = slice(h * DH, (h + 1) * DH)
        s = jnp.dot(q_ref[:, sl], k_ref[:, sl].T,
                    preferred_element_type=jnp.float32) * SCALE
        m = jnp.max(s, axis=-1, keepdims=True)
        e = jnp.exp(s - m)
        o = jnp.dot(e, v_ref[:, sl], preferred_element_type=jnp.float32)
        o_ref[:, sl] = o / jnp.sum(e, axis=-1, keepdims=True)


def _sparse_kernel(q_ref, k_ref, v_ref, o_ref):
    for h in range(H):
        sl = slice(h * DH, (h + 1) * DH)
        s = jnp.dot(q_ref[:, sl], k_ref[:, sl].T,
                    preferred_element_type=jnp.float32) * SCALE
        cur = s
        rowmax = None
        for i in range(KS - 1):
            m = jnp.max(cur, axis=-1, keepdims=True)
            if i == 0:
                rowmax = m
            cur = jnp.where(cur >= m, NEG, cur)
        thr = jnp.max(cur, axis=-1, keepdims=True)
        e = jnp.where(s >= thr, jnp.exp(s - rowmax), 0.0)
        o = jnp.dot(e, v_ref[:, sl], preferred_element_type=jnp.float32)
        o_ref[:, sl] = o / jnp.sum(e, axis=-1, keepdims=True)


def _perf_kernel(q_ref, k_ref, v_ref, f_ref, o_ref):
    def elu1(t):
        return jnp.where(t > 0, t + 1.0, jnp.exp(jnp.minimum(t, 0.0)))

    for h in range(H):
        sl = slice(h * DH, (h + 1) * DH)
        f = f_ref[h]
        qf = elu1(jnp.dot(q_ref[:, sl], f,
                          preferred_element_type=jnp.float32))
        kf = elu1(jnp.dot(k_ref[:, sl], f,
                          preferred_element_type=jnp.float32))
        v = v_ref[:, sl]
        kv = jax.lax.dot_general(kf, v, (((0,), (0,)), ((), ())),
                                 preferred_element_type=jnp.float32)
        ksum = jnp.sum(kf, axis=0, keepdims=True)
        z = 1.0 / (jnp.sum(qf * ksum, axis=1, keepdims=True) + 1e-6)
        o_ref[:, sl] = jnp.dot(qf, kv,
                               preferred_element_type=jnp.float32) * z


def _wts_kernel(l_ref, o_ref):
    l = l_ref[:, :E]
    m1 = jnp.max(l, axis=-1, keepdims=True)
    l2 = jnp.where(l >= m1, NEG, l)
    m2 = jnp.max(l2, axis=-1, keepdims=True)
    e2 = jnp.exp(m2 - m1)
    den = 1.0 + e2
    g1 = 1.0 / den
    g2 = e2 / den
    o_ref[...] = jnp.where(l >= m1, g1, jnp.where(l >= m2, g2, 0.0))


def _expert_kernel(ctx_ref, wts_ref, wo_ref, bo_ref, o_ref):
    ctx = ctx_ref[...]
    w = wts_ref[...]
    total = jnp.zeros(o_ref.shape, jnp.float32)
    for e in range(E):
        total += w[:, e:e + 1] * (
            jnp.dot(ctx, wo_ref[e], preferred_element_type=jnp.float32)
            + bo_ref[e])
    o_ref[...] = total


def _taa_ln1_kernel(g_ref, sp_ref, pf_ref, wsp_ref, wpf_ref,
                    x_ref, moe_ref, lg_ref, lb_ref, o_ref):
    g0 = g_ref[0:1, 2:3]
    g1 = g_ref[0:1, 3:4]
    taa = (jnp.dot(sp_ref[...] * g0, wsp_ref[...],
                   preferred_element_type=jnp.float32)
           + jnp.dot(pf_ref[...] * g1, wpf_ref[...],
                     preferred_element_type=jnp.float32))
    aw = g_ref[0:1, 0:1]
    ag0 = g_ref[0:1, 4:5]
    ag1 = g_ref[0:1, 5:6]
    att = aw * (ag0 * moe_ref[...] + ag1 * taa) * DEPTH
    y = x_ref[...] + att
    m = jnp.mean(y, axis=-1, keepdims=True)
    v = jnp.mean((y - m) ** 2, axis=-1, keepdims=True)
    o_ref[...] = (y - m) / jnp.sqrt(v + 1e-5) * lg_ref[...] + lb_ref[...]


def _ffn_kernel(g_ref, y_ref, w1_ref, b1_ref, w2_ref, b2_ref,
                lg_ref, lb_ref, o_ref):
    fw = g_ref[0:1, 1:2]
    y = y_ref[...]
    h = jax.nn.gelu(jnp.dot(y, w1_ref[...],
                            preferred_element_type=jnp.float32) + b1_ref[...])
    z = jnp.dot(h, w2_ref[...], preferred_element_type=jnp.float32) + b2_ref[...]
    t = y + fw * z * DEPTH
    m = jnp.mean(t, axis=-1, keepdims=True)
    v = jnp.mean((t - m) ** 2, axis=-1, keepdims=True)
    o_ref[...] = (t - m) / jnp.sqrt(v + 1e-5) * lg_ref[...] + lb_ref[...]


def kernel(x, arg_w1, arg_b1, arg_w2, arg_b2, moe_wq, moe_wk, moe_wv,
           moe_router_w, moe_router_b, moe_wo, moe_bo, sp_wq, sp_wk, sp_wv,
           sp_wo, pf_wq, pf_wk, pf_wv, pf_wo, pf_feat, taa_gate_w, taa_gate_b,
           attn_gate_w, attn_gate_b, ffn_w1, ffn_b1, ffn_w2, ffn_b2,
           ln1_g, ln1_b, ln2_g, ln2_b):
    xs = x.reshape(S, D)

    gates = pl.pallas_call(
        _gates_kernel,
        out_shape=jax.ShapeDtypeStruct((1, 128), jnp.float32),
    )(xs, arg_w1, arg_b1.reshape(1, 128), arg_w2, arg_b2.reshape(1, 2),
      taa_gate_w, taa_gate_b.reshape(1, 2), attn_gate_w,
      attn_gate_b.reshape(1, 2))

    wcat = jnp.concatenate(
        [moe_wq, moe_wk, moe_wv, sp_wq, sp_wk, sp_wv, pf_wq, pf_wk, pf_wv,
         jnp.pad(moe_router_w, ((0, 0), (0, 128 - E)))], axis=1)
    bcat = jnp.concatenate(
        [jnp.zeros((9 * D,), jnp.float32),
         jnp.pad(moe_router_b, (0, 128 - E))]).reshape(1, NPROJ)
    proj = pl.pallas_call(
        _proj_kernel,
        grid=(NPROJ // NB,),
        in_specs=[
            pl.BlockSpec((S, D), lambda j: (0, 0)),
            pl.BlockSpec((D, NB), lambda j: (0, j)),
            pl.BlockSpec((1, NB), lambda j: (0, j)),
        ],
        out_specs=pl.BlockSpec((S, NB), lambda j: (0, j)),
        out_shape=jax.ShapeDtypeStruct((S, NPROJ), jnp.float32),
    )(xs, wcat, bcat)

    def attn(fn, qb, kb, vb):
        return pl.pallas_call(
            fn,
            grid=(S // BA,),
            in_specs=[
                pl.BlockSpec((BA, D), lambda i: (i, qb)),
                pl.BlockSpec((S, D), lambda i: (0, kb)),
                pl.BlockSpec((S, D), lambda i: (0, vb)),
            ],
            out_specs=pl.BlockSpec((BA, D), lambda i: (i, 0)),
            out_shape=jax.ShapeDtypeStruct((S, D), jnp.float32),
        )(proj, proj, proj)

    moe_ctx = attn(_attn_kernel, 0, 1, 2)
    sp_ctx = attn(_sparse_kernel, 3, 4, 5)

    pf_ctx = pl.pallas_call(
        _perf_kernel,
        grid=(1,),
        in_specs=[
            pl.BlockSpec((S, D), lambda i: (0, 6)),
            pl.BlockSpec((S, D), lambda i: (0, 7)),
            pl.BlockSpec((S, D), lambda i: (0, 8)),
            pl.BlockSpec((H, DH, KD), lambda i: (0, 0, 0)),
        ],
        out_specs=pl.BlockSpec((S, D), lambda i: (0, 0)),
        out_shape=jax.ShapeDtypeStruct((S, D), jnp.float32),
    )(proj, proj, proj, pf_feat)

    wts = pl.pallas_call(
        _wts_kernel,
        grid=(1,),
        in_specs=[pl.BlockSpec((S, 128), lambda i: (0, 54))],
        out_specs=pl.BlockSpec((S, E), lambda i: (0, 0)),
        out_shape=jax.ShapeDtypeStruct((S, E), jnp.float32),
    )(proj)

    moe_out = pl.pallas_call(
        _expert_kernel,
        grid=(S // BS,),
        in_specs=[
            pl.BlockSpec((BS, D), lambda i: (i, 0)),
            pl.BlockSpec((BS, E), lambda i: (i, 0)),
            pl.BlockSpec((E, D, D), lambda i: (0, 0, 0)),
            pl.BlockSpec((E, 1, D), lambda i: (0, 0, 0)),
        ],
        out_specs=pl.BlockSpec((BS, D), lambda i: (i, 0)),
        out_shape=jax.ShapeDtypeStruct((S, D), jnp.float32),
    )(moe_ctx, wts, moe_wo, moe_bo.reshape(E, 1, D))

    y = pl.pallas_call(
        _taa_ln1_kernel,
        grid=(S // BS,),
        in_specs=[
            pl.BlockSpec((1, 128), lambda i: (0, 0)),
            pl.BlockSpec((BS, D), lambda i: (i, 0)),
            pl.BlockSpec((BS, D), lambda i: (i, 0)),
            pl.BlockSpec((D, D), lambda i: (0, 0)),
            pl.BlockSpec((D, D), lambda i: (0, 0)),
            pl.BlockSpec((BS, D), lambda i: (i, 0)),
            pl.BlockSpec((BS, D), lambda i: (i, 0)),
            pl.BlockSpec((1, D), lambda i: (0, 0)),
            pl.BlockSpec((1, D), lambda i: (0, 0)),
        ],
        out_specs=pl.BlockSpec((BS, D), lambda i: (i, 0)),
        out_shape=jax.ShapeDtypeStruct((S, D), jnp.float32),
    )(gates, sp_ctx, pf_ctx, sp_wo, pf_wo, xs, moe_out,
      ln1_g.reshape(1, D), ln1_b.reshape(1, D))

    out = pl.pallas_call(
        _ffn_kernel,
        grid=(S // BS,),
        in_specs=[
            pl.BlockSpec((1, 128), lambda i: (0, 0)),
            pl.BlockSpec((BS, D), lambda i: (i, 0)),
            pl.BlockSpec((D, FF), lambda i: (0, 0)),
            pl.BlockSpec((1, FF), lambda i: (0, 0)),
            pl.BlockSpec((FF, D), lambda i: (0, 0)),
            pl.BlockSpec((1, D), lambda i: (0, 0)),
            pl.BlockSpec((1, D), lambda i: (0, 0)),
            pl.BlockSpec((1, D), lambda i: (0, 0)),
        ],
        out_specs=pl.BlockSpec((BS, D), lambda i: (i, 0)),
        out_shape=jax.ShapeDtypeStruct((S, D), jnp.float32),
    )(gates, y, ffn_w1, ffn_b1.reshape(1, FF), ffn_w2,
      ffn_b2.reshape(1, D), ln2_g.reshape(1, D), ln2_b.reshape(1, D))

    return out.reshape(1, S, D)


# speedup vs baseline: 1.0857x; 1.0857x over previous
import numpy as np
import jax
import jax.numpy as jnp
from jax.experimental import pallas as pl

D = 768
H = 12
DH = 64
E = 8
FF = 3072
KD = 256
KS = 10
S = 2048
DEPTH = 1.25
SCALE = 1.0 / np.sqrt(DH)
BS = 512
BA = 256
NPROJ = 9 * D + 128
NB = 640
NEG = -1e30


def _f32(*dots):
    return dict(preferred_element_type=jnp.float32)


def _gates_kernel(x_ref, aw1_ref, ab1_ref, aw2_ref, ab2_ref,
                  tw_ref, tb_ref, gw_ref, gb_ref, o_ref):
    avg = jnp.mean(x_ref[...], axis=0, keepdims=True)
    h = jax.nn.gelu(jnp.dot(avg, aw1_ref[...],
                            preferred_element_type=jnp.float32) + ab1_ref[...])
    gwv = jax.nn.sigmoid(jnp.dot(h, aw2_ref[...],
                                 preferred_element_type=jnp.float32) + ab2_ref[...])
    tg = jax.nn.softmax(jnp.dot(avg, tw_ref[...],
                                preferred_element_type=jnp.float32) + tb_ref[...], -1)
    ag = jax.nn.softmax(jnp.dot(avg, gw_ref[...],
                                preferred_element_type=jnp.float32) + gb_ref[...], -1)
    vals = jnp.concatenate(
        [gwv, tg, ag, jnp.zeros((1, 122), jnp.float32)], axis=1)
    o_ref[...] = vals


def _proj_kernel(x_ref, w_ref, b_ref, o_ref):
    o_ref[...] = jnp.dot(x_ref[...], w_ref[...],
                         preferred_element_type=jnp.float32) + b_ref[...]


def _attn_kernel(q_ref, k_ref, v_ref, o_ref):
    for h in range(H):
        sl = slice(h * DH, (h + 1) * DH)
        s = jnp.dot(q_ref[:, sl], k_ref[:, sl].T,
                    preferred_element_type=jnp.float32) * SCALE
        m = jnp.max(s, axis=-1, keepdims=True)
        e = jnp.exp(s - m)
        o = jnp.dot(e, v_ref[:, sl], preferred_element_type=jnp.float32)
        o_ref[:, sl] = o / jnp.sum(e, axis=-1, keepdims=True)


def _sparse_kernel(q_ref, k_ref, v_ref, o_ref):
    for h in range(H):
        sl = slice(h * DH, (h + 1) * DH)
        s = jnp.dot(q_ref[:, sl], k_ref[:, sl].T,
                    preferred_element_type=jnp.float32) * SCALE
        cur = s
        rowmax = None
        for i in range(KS - 1):
            m = jnp.max(cur, axis=-1, keepdims=True)
            if i == 0:
                rowmax = m
            cur = jnp.where(cur >= m, NEG, cur)
        thr = jnp.max(cur, axis=-1, keepdims=True)
        e = jnp.where(s >= thr, jnp.exp(s - rowmax), 0.0)
        o = jnp.dot(e, v_ref[:, sl], preferred_element_type=jnp.float32)
        o_ref[:, sl] = o / jnp.sum(e, axis=-1, keepdims=True)


def _perf_kernel(q_ref, k_ref, v_ref, f_ref, o_ref):
    def elu1(t):
        return jnp.where(t > 0, t + 1.0, jnp.exp(jnp.minimum(t, 0.0)))

    for h in range(H):
        sl = slice(h * DH, (h + 1) * DH)
        f = f_ref[h]
        qf = elu1(jnp.dot(q_ref[:, sl], f,
                          preferred_element_type=jnp.float32))
        kf = elu1(jnp.dot(k_ref[:, sl], f,
                          preferred_element_type=jnp.float32))
        v = v_ref[:, sl]
        kv = jax.lax.dot_general(kf, v, (((0,), (0,)), ((), ())),
                                 preferred_element_type=jnp.float32)
        ksum = jnp.sum(kf, axis=0, keepdims=True)
        z = 1.0 / (jnp.sum(qf * ksum, axis=1, keepdims=True) + 1e-6)
        o_ref[:, sl] = jnp.dot(qf, kv,
                               preferred_element_type=jnp.float32) * z


def _wts_kernel(l_ref, o_ref):
    l = l_ref[:, :E]
    m1 = jnp.max(l, axis=-1, keepdims=True)
    l2 = jnp.where(l >= m1, NEG, l)
    m2 = jnp.max(l2, axis=-1, keepdims=True)
    e2 = jnp.exp(m2 - m1)
    den = 1.0 + e2
    g1 = 1.0 / den
    g2 = e2 / den
    o_ref[...] = jnp.where(l >= m1, g1, jnp.where(l >= m2, g2, 0.0))


def _expert_kernel(ctx_ref, wts_ref, wo_ref, bo_ref, o_ref):
    ctx = ctx_ref[...]
    w = wts_ref[...]
    total = jnp.zeros(o_ref.shape, jnp.float32)
    for e in range(E):
        total += w[:, e:e + 1] * (
            jnp.dot(ctx, wo_ref[e], preferred_element_type=jnp.float32)
            + bo_ref[e])
    o_ref[...] = total


def _taa_ln1_kernel(g_ref, sp_ref, pf_ref, wsp_ref, wpf_ref,
                    x_ref, moe_ref, lg_ref, lb_ref, o_ref):
    g0 = g_ref[0:1, 2:3]
    g1 = g_ref[0:1, 3:4]
    taa = (jnp.dot(sp_ref[...] * g0, wsp_ref[...],
                   preferred_element_type=jnp.float32)
           + jnp.dot(pf_ref[...] * g1, wpf_ref[...],
                     preferred_element_type=jnp.float32))
    aw = g_ref[0:1, 0:1]
    ag0 = g_ref[0:1, 4:5]
    ag1 = g_ref[0:1, 5:6]
    att = aw * (ag0 * moe_ref[...] + ag1 * taa) * DEPTH
    y = x_ref[...] + att
    m = jnp.mean(y, axis=-1, keepdims=True)
    v = jnp.mean((y - m) ** 2, axis=-1, keepdims=True)
    o_ref[...] = (y - m) / jnp.sqrt(v + 1e-5) * lg_ref[...] + lb_ref[...]


def _ffn_kernel(g_ref, y_ref, w1_ref, b1_ref, w2_ref, b2_ref,
                lg_ref, lb_ref, o_ref):
    fw = g_ref[0:1, 1:2]
    y = y_ref[...]
    h = jax.nn.gelu(jnp.dot(y, w1_ref[...],
                            preferred_element_type=jnp.float32) + b1_ref[...])
    z = jnp.dot(h, w2_ref[...], preferred_element_type=jnp.float32) + b2_ref[...]
    t = y + fw * z * DEPTH
    m = jnp.mean(t, axis=-1, keepdims=True)
    v = jnp.mean((t - m) ** 2, axis=-1, keepdims=True)
    o_ref[...] = (t - m) / jnp.sqrt(v + 1e-5) * lg_ref[...] + lb_ref[...]


def kernel(x, arg_w1, arg_b1, arg_w2, arg_b2, moe_wq, moe_wk, moe_wv,
           moe_router_w, moe_router_b, moe_wo, moe_bo, sp_wq, sp_wk, sp_wv,
           sp_wo, pf_wq, pf_wk, pf_wv, pf_wo, pf_feat, taa_gate_w, taa_gate_b,
           attn_gate_w, attn_gate_b, ffn_w1, ffn_b1, ffn_w2, ffn_b2,
           ln1_g, ln1_b, ln2_g, ln2_b):
    xs = x.reshape(S, D)

    gates = pl.pallas_call(
        _gates_kernel,
        out_shape=jax.ShapeDtypeStruct((1, 128), jnp.float32),
    )(xs, arg_w1, arg_b1.reshape(1, 128), arg_w2, arg_b2.reshape(1, 2),
      taa_gate_w, taa_gate_b.reshape(1, 2), attn_gate_w,
      attn_gate_b.reshape(1, 2))

    wcat = jnp.concatenate(
        [moe_wq, moe_wk, moe_wv, sp_wq, sp_wk, sp_wv, pf_wq, pf_wk, pf_wv,
         jnp.pad(moe_router_w, ((0, 0), (0, 128 - E)))], axis=1)
    bcat = jnp.concatenate(
        [jnp.zeros((9 * D,), jnp.float32),
         jnp.pad(moe_router_b, (0, 128 - E))]).reshape(1, NPROJ)
    proj = pl.pallas_call(
        _proj_kernel,
        grid=(NPROJ // NB,),
        in_specs=[
            pl.BlockSpec((S, D), lambda j: (0, 0)),
            pl.BlockSpec((D, NB), lambda j: (0, j)),
            pl.BlockSpec((1, NB), lambda j: (0, j)),
        ],
        out_specs=pl.BlockSpec((S, NB), lambda j: (0, j)),
        out_shape=jax.ShapeDtypeStruct((S, NPROJ), jnp.float32),
    )(xs, wcat, bcat)

    def attn(fn, qb, kb, vb):
        return pl.pallas_call(
            fn,
            grid=(S // BA,),
            in_specs=[
                pl.BlockSpec((BA, D), lambda i: (i, qb)),
                pl.BlockSpec((S, D), lambda i: (0, kb)),
                pl.BlockSpec((S, D), lambda i: (0, vb)),
            ],
            out_specs=pl.BlockSpec((BA, D), lambda i: (i, 0)),
            out_shape=jax.ShapeDtypeStruct((S, D), jnp.float32),
        )(proj, proj, proj)

    moe_ctx = attn(_attn_kernel, 0, 1, 2)
    sp_ctx = attn(_sparse_kernel, 3, 4, 5)

    pf_ctx = pl.pallas_call(
        _perf_kernel,
        grid=(1,),
        in_specs=[
            pl.BlockSpec((S, D), lambda i: (0, 6)),
            pl.BlockSpec((S, D), lambda i: (0, 7)),
            pl.BlockSpec((S, D), lambda i: (0, 8)),
            pl.BlockSpec((H, DH, KD), lambda i: (0, 0, 0)),
        ],
        out_specs=pl.BlockSpec((S, D), lambda i: (0, 0)),
        out_shape=jax.ShapeDtypeStruct((S, D), jnp.float32),
    )(proj, proj, proj, pf_feat)

    wts = pl.pallas_call(
        _wts_kernel,
        grid=(1,),
        in_specs=[pl.BlockSpec((S, 128), lambda i: (0, 54))],
        out_specs=pl.BlockSpec((S, E), lambda i: (0, 0)),
        out_shape=jax.ShapeDtypeStruct((S, E), jnp.float32),
    )(proj)

    moe_out = pl.pallas_call(
        _expert_kernel,
        grid=(S // BS,),
        in_specs=[
            pl.BlockSpec((BS, D), lambda i: (i, 0)),
            pl.BlockSpec((BS, E), lambda i: (i, 0)),
            pl.BlockSpec((E, D, D), lambda i: (0, 0, 0)),
            pl.BlockSpec((E, 1, D), lambda i: (0, 0, 0)),
        ],
        out_specs=pl.BlockSpec((BS, D), lambda i: (i, 0)),
        out_shape=jax.ShapeDtypeStruct((S, D), jnp.float32),
    )(moe_ctx, wts, moe_wo, moe_bo.reshape(E, 1, D))

    y = pl.pallas_call(
        _taa_ln1_kernel,
        grid=(S // BS,),
        in_specs=[
            pl.BlockSpec((1, 128), lambda i: (0, 0)),
            pl.BlockSpec((BS, D), lambda i: (i, 0)),
            pl.BlockSpec((BS, D), lambda i: (i, 0)),
            pl.BlockSpec((D, D), lambda i: (0, 0)),
            pl.BlockSpec((D, D), lambda i: (0, 0)),
            pl.BlockSpec((BS, D), lambda i: (i, 0)),
            pl.BlockSpec((BS, D), lambda i: (i, 0)),
            pl.BlockSpec((1, D), lambda i: (0, 0)),
            pl.BlockSpec((1, D), lambda i: (0, 0)),
        ],
        out_specs=pl.BlockSpec((BS, D), lambda i: (i, 0)),
        out_shape=jax.ShapeDtypeStruct((S, D), jnp.float32),
    )(gates, sp_ctx, pf_ctx, sp_wo, pf_wo, xs, moe_out,
      ln1_g.reshape(1, D), ln1_b.reshape(1, D))

    out = pl.pallas_call(
        _ffn_kernel,
        grid=(S // BS,),
        in_specs=[
            pl.BlockSpec((1, 128), lambda i: (0, 0)),
            pl.BlockSpec((BS, D), lambda i: (i, 0)),
            pl.BlockSpec((D, FF), lambda i: (0, 0)),
            pl.BlockSpec((1, FF), lambda i: (0, 0)),
            pl.BlockSpec((FF, D), lambda i: (0, 0)),
            pl.BlockSpec((1, D), lambda i: (0, 0)),
            pl.BlockSpec((1, D), lambda i: (0, 0)),
            pl.BlockSpec((1, D), lambda i: (0, 0)),
        ],
        out_specs=pl.BlockSpec((BS, D), lambda i: (i, 0)),
        out_shape=jax.ShapeDtypeStruct((S, D), jnp.float32),
    )(gates, y, ffn_w1, ffn_b1.reshape(1, FF), ffn_w2,
      ffn_b2.reshape(1, D), ln2_g.reshape(1, D), ln2_b.reshape(1, D))

    return out.reshape(1, S, D)


# speedup vs baseline: 1.0909x; 1.0048x over previous
import numpy as np
import jax
import jax.numpy as jnp
from jax.experimental import pallas as pl

D = 768
H = 12
DH = 64
E = 8
FF = 3072
KD = 256
KS = 10
S = 2048
DEPTH = 1.25
SCALE = 1.0 / np.sqrt(DH)
BS = 512
BA = 256
NPROJ = 9 * D + 128
NB = 1408
NEG = -1e30


def _f32(*dots):
    return dict(preferred_element_type=jnp.float32)


def _gates_kernel(x_ref, aw1_ref, ab1_ref, aw2_ref, ab2_ref,
                  tw_ref, tb_ref, gw_ref, gb_ref, o_ref):
    avg = jnp.mean(x_ref[...], axis=0, keepdims=True)
    h = jax.nn.gelu(jnp.dot(avg, aw1_ref[...],
                            preferred_element_type=jnp.float32) + ab1_ref[...])
    gwv = jax.nn.sigmoid(jnp.dot(h, aw2_ref[...],
                                 preferred_element_type=jnp.float32) + ab2_ref[...])
    tg = jax.nn.softmax(jnp.dot(avg, tw_ref[...],
                                preferred_element_type=jnp.float32) + tb_ref[...], -1)
    ag = jax.nn.softmax(jnp.dot(avg, gw_ref[...],
                                preferred_element_type=jnp.float32) + gb_ref[...], -1)
    vals = jnp.concatenate(
        [gwv, tg, ag, jnp.zeros((1, 122), jnp.float32)], axis=1)
    o_ref[...] = vals


def _proj_kernel(x_ref, w_ref, b_ref, o_ref):
    o_ref[...] = jnp.dot(x_ref[...], w_ref[...],
                         preferred_element_type=jnp.float32) + b_ref[...]


def _attn_kernel(q_ref, k_ref, v_ref, o_ref):
    for h in range(H):
        sl = slice(h * DH, (h + 1) * DH)
        s = jnp.dot(q_ref[:, sl], k_ref[:, sl].T,
                    preferred_element_type=jnp.float32) * SCALE
        m = jnp.max(s, axis=-1, keepdims=True)
        e = jnp.exp(s - m)
        o = jnp.dot(e, v_ref[:, sl], preferred_element_type=jnp.float32)
        o_ref[:, sl] = o / jnp.sum(e, axis=-1, keepdims=True)


def _sparse_kernel(q_ref, k_ref, v_ref, o_ref):
    for h in range(H):
        sl = slice(h * DH, (h + 1) * DH)
        s = jnp.dot(q_ref[:, sl], k_ref[:, sl].T,
                    preferred_element_type=jnp.float32) * SCALE
        cur = s
        rowmax = None
        for i in range(KS - 1):
            m = jnp.max(cur, axis=-1, keepdims=True)
            if i == 0:
                rowmax = m
            cur = jnp.where(cur >= m, NEG, cur)
        thr = jnp.max(cur, axis=-1, keepdims=True)
        e = jnp.where(s >= thr, jnp.exp(s - rowmax), 0.0)
        o = jnp.dot(e, v_ref[:, sl], preferred_element_type=jnp.float32)
        o_ref[:, sl] = o / jnp.sum(e, axis=-1, keepdims=True)


def _perf_kernel(q_ref, k_ref, v_ref, f_ref, o_ref):
    def elu1(t):
        return jnp.where(t > 0, t + 1.0, jnp.exp(jnp.minimum(t, 0.0)))

    for h in range(H):
        sl = slice(h * DH, (h + 1) * DH)
        f = f_ref[h]
        qf = elu1(jnp.dot(q_ref[:, sl], f,
                          preferred_element_type=jnp.float32))
        kf = elu1(jnp.dot(k_ref[:, sl], f,
                          preferred_element_type=jnp.float32))
        v = v_ref[:, sl]
        kv = jax.lax.dot_general(kf, v, (((0,), (0,)), ((), ())),
                                 preferred_element_type=jnp.float32)
        ksum = jnp.sum(kf, axis=0, keepdims=True)
        z = 1.0 / (jnp.sum(qf * ksum, axis=1, keepdims=True) + 1e-6)
        o_ref[:, sl] = jnp.dot(qf, kv,
                               preferred_element_type=jnp.float32) * z


def _wts_kernel(l_ref, o_ref):
    l = l_ref[:, :E]
    m1 = jnp.max(l, axis=-1, keepdims=True)
    l2 = jnp.where(l >= m1, NEG, l)
    m2 = jnp.max(l2, axis=-1, keepdims=True)
    e2 = jnp.exp(m2 - m1)
    den = 1.0 + e2
    g1 = 1.0 / den
    g2 = e2 / den
    o_ref[...] = jnp.where(l >= m1, g1, jnp.where(l >= m2, g2, 0.0))


def _expert_kernel(ctx_ref, wts_ref, wo_ref, bo_ref, o_ref):
    ctx = ctx_ref[...]
    w = wts_ref[...]
    total = jnp.zeros(o_ref.shape, jnp.float32)
    for e in range(E):
        total += w[:, e:e + 1] * (
            jnp.dot(ctx, wo_ref[e], preferred_element_type=jnp.float32)
            + bo_ref[e])
    o_ref[...] = total


def _taa_ln1_kernel(g_ref, sp_ref, pf_ref, wsp_ref, wpf_ref,
                    x_ref, moe_ref, lg_ref, lb_ref, o_ref):
    g0 = g_ref[0:1, 2:3]
    g1 = g_ref[0:1, 3:4]
    taa = (jnp.dot(sp_ref[...] * g0, wsp_ref[...],
                   preferred_element_type=jnp.float32)
           + jnp.dot(pf_ref[...] * g1, wpf_ref[...],
                     preferred_element_type=jnp.float32))
    aw = g_ref[0:1, 0:1]
    ag0 = g_ref[0:1, 4:5]
    ag1 = g_ref[0:1, 5:6]
    att = aw * (ag0 * moe_ref[...] + ag1 * taa) * DEPTH
    y = x_ref[...] + att
    m = jnp.mean(y, axis=-1, keepdims=True)
    v = jnp.mean((y - m) ** 2, axis=-1, keepdims=True)
    o_ref[...] = (y - m) / jnp.sqrt(v + 1e-5) * lg_ref[...] + lb_ref[...]


def _ffn_kernel(g_ref, y_ref, w1_ref, b1_ref, w2_ref, b2_ref,
                lg_ref, lb_ref, o_ref):
    fw = g_ref[0:1, 1:2]
    y = y_ref[...]
    h = jax.nn.gelu(jnp.dot(y, w1_ref[...],
                            preferred_element_type=jnp.float32) + b1_ref[...])
    z = jnp.dot(h, w2_ref[...], preferred_element_type=jnp.float32) + b2_ref[...]
    t = y + fw * z * DEPTH
    m = jnp.mean(t, axis=-1, keepdims=True)
    v = jnp.mean((t - m) ** 2, axis=-1, keepdims=True)
    o_ref[...] = (t - m) / jnp.sqrt(v + 1e-5) * lg_ref[...] + lb_ref[...]


def kernel(x, arg_w1, arg_b1, arg_w2, arg_b2, moe_wq, moe_wk, moe_wv,
           moe_router_w, moe_router_b, moe_wo, moe_bo, sp_wq, sp_wk, sp_wv,
           sp_wo, pf_wq, pf_wk, pf_wv, pf_wo, pf_feat, taa_gate_w, taa_gate_b,
           attn_gate_w, attn_gate_b, ffn_w1, ffn_b1, ffn_w2, ffn_b2,
           ln1_g, ln1_b, ln2_g, ln2_b):
    xs = x.reshape(S, D)

    gates = pl.pallas_call(
        _gates_kernel,
        out_shape=jax.ShapeDtypeStruct((1, 128), jnp.float32),
    )(xs, arg_w1, arg_b1.reshape(1, 128), arg_w2, arg_b2.reshape(1, 2),
      taa_gate_w, taa_gate_b.reshape(1, 2), attn_gate_w,
      attn_gate_b.reshape(1, 2))

    wcat = jnp.concatenate(
        [moe_wq, moe_wk, moe_wv, sp_wq, sp_wk, sp_wv, pf_wq, pf_wk, pf_wv,
         jnp.pad(moe_router_w, ((0, 0), (0, 128 - E)))], axis=1)
    bcat = jnp.concatenate(
        [jnp.zeros((9 * D,), jnp.float32),
         jnp.pad(moe_router_b, (0, 128 - E))]).reshape(1, NPROJ)
    proj = pl.pallas_call(
        _proj_kernel,
        grid=(NPROJ // NB,),
        in_specs=[
            pl.BlockSpec((S, D), lambda j: (0, 0)),
            pl.BlockSpec((D, NB), lambda j: (0, j)),
            pl.BlockSpec((1, NB), lambda j: (0, j)),
        ],
        out_specs=pl.BlockSpec((S, NB), lambda j: (0, j)),
        out_shape=jax.ShapeDtypeStruct((S, NPROJ), jnp.float32),
    )(xs, wcat, bcat)

    def attn(fn, qb, kb, vb):
        return pl.pallas_call(
            fn,
            grid=(S // BA,),
            in_specs=[
                pl.BlockSpec((BA, D), lambda i: (i, qb)),
                pl.BlockSpec((S, D), lambda i: (0, kb)),
                pl.BlockSpec((S, D), lambda i: (0, vb)),
            ],
            out_specs=pl.BlockSpec((BA, D), lambda i: (i, 0)),
            out_shape=jax.ShapeDtypeStruct((S, D), jnp.float32),
        )(proj, proj, proj)

    moe_ctx = attn(_attn_kernel, 0, 1, 2)
    sp_ctx = attn(_sparse_kernel, 3, 4, 5)

    pf_ctx = pl.pallas_call(
        _perf_kernel,
        grid=(1,),
        in_specs=[
            pl.BlockSpec((S, D), lambda i: (0, 6)),
            pl.BlockSpec((S, D), lambda i: (0, 7)),
            pl.BlockSpec((S, D), lambda i: (0, 8)),
            pl.BlockSpec((H, DH, KD), lambda i: (0, 0, 0)),
        ],
        out_specs=pl.BlockSpec((S, D), lambda i: (0, 0)),
        out_shape=jax.ShapeDtypeStruct((S, D), jnp.float32),
    )(proj, proj, proj, pf_feat)

    wts = pl.pallas_call(
        _wts_kernel,
        grid=(1,),
        in_specs=[pl.BlockSpec((S, 128), lambda i: (0, 54))],
        out_specs=pl.BlockSpec((S, E), lambda i: (0, 0)),
        out_shape=jax.ShapeDtypeStruct((S, E), jnp.float32),
    )(proj)

    moe_out = pl.pallas_call(
        _expert_kernel,
        grid=(S // BS,),
        in_specs=[
            pl.BlockSpec((BS, D), lambda i: (i, 0)),
            pl.BlockSpec((BS, E), lambda i: (i, 0)),
            pl.BlockSpec((E, D, D), lambda i: (0, 0, 0)),
            pl.BlockSpec((E, 1, D), lambda i: (0, 0, 0)),
        ],
        out_specs=pl.BlockSpec((BS, D), lambda i: (i, 0)),
        out_shape=jax.ShapeDtypeStruct((S, D), jnp.float32),
    )(moe_ctx, wts, moe_wo, moe_bo.reshape(E, 1, D))

    y = pl.pallas_call(
        _taa_ln1_kernel,
        grid=(S // BS,),
        in_specs=[
            pl.BlockSpec((1, 128), lambda i: (0, 0)),
            pl.BlockSpec((BS, D), lambda i: (i, 0)),
            pl.BlockSpec((BS, D), lambda i: (i, 0)),
            pl.BlockSpec((D, D), lambda i: (0, 0)),
            pl.BlockSpec((D, D), lambda i: (0, 0)),
            pl.BlockSpec((BS, D), lambda i: (i, 0)),
            pl.BlockSpec((BS, D), lambda i: (i, 0)),
            pl.BlockSpec((1, D), lambda i: (0, 0)),
            pl.BlockSpec((1, D), lambda i: (0, 0)),
        ],
        out_specs=pl.BlockSpec((BS, D), lambda i: (i, 0)),
        out_shape=jax.ShapeDtypeStruct((S, D), jnp.float32),
    )(gates, sp_ctx, pf_ctx, sp_wo, pf_wo, xs, moe_out,
      ln1_g.reshape(1, D), ln1_b.reshape(1, D))

    out = pl.pallas_call(
        _ffn_kernel,
        grid=(S // BS,),
        in_specs=[
            pl.BlockSpec((1, 128), lambda i: (0, 0)),
            pl.BlockSpec((BS, D), lambda i: (i, 0)),
            pl.BlockSpec((D, FF), lambda i: (0, 0)),
            pl.BlockSpec((1, FF), lambda i: (0, 0)),
            pl.BlockSpec((FF, D), lambda i: (0, 0)),
            pl.BlockSpec((1, D), lambda i: (0, 0)),
            pl.BlockSpec((1, D), lambda i: (0, 0)),
            pl.BlockSpec((1, D), lambda i: (0, 0)),
        ],
        out_specs=pl.BlockSpec((BS, D), lambda i: (i, 0)),
        out_shape=jax.ShapeDtypeStruct((S, D), jnp.float32),
    )(gates, y, ffn_w1, ffn_b1.reshape(1, FF), ffn_w2,
      ffn_b2.reshape(1, D), ln2_g.reshape(1, D), ln2_b.reshape(1, D))

    return out.reshape(1, S, D)


# speedup vs baseline: 1.0960x; 1.0047x over previous
import numpy as np
import jax
import jax.numpy as jnp
from jax.experimental import pallas as pl

D = 768
H = 12
DH = 64
E = 8
FF = 3072
KD = 256
KS = 10
S = 2048
DEPTH = 1.25
SCALE = 1.0 / np.sqrt(DH)
BS = 512
BA = 256
NPROJ = 9 * D + 128
NB = 1408
NEG = -1e30


def _f32(*dots):
    return dict(preferred_element_type=jnp.float32)


def _gates_kernel(x_ref, aw1_ref, ab1_ref, aw2_ref, ab2_ref,
                  tw_ref, tb_ref, gw_ref, gb_ref, o_ref):
    avg = jnp.mean(x_ref[...], axis=0, keepdims=True)
    h = jax.nn.gelu(jnp.dot(avg, aw1_ref[...],
                            preferred_element_type=jnp.float32) + ab1_ref[...])
    gwv = jax.nn.sigmoid(jnp.dot(h, aw2_ref[...],
                                 preferred_element_type=jnp.float32) + ab2_ref[...])
    tg = jax.nn.softmax(jnp.dot(avg, tw_ref[...],
                                preferred_element_type=jnp.float32) + tb_ref[...], -1)
    ag = jax.nn.softmax(jnp.dot(avg, gw_ref[...],
                                preferred_element_type=jnp.float32) + gb_ref[...], -1)
    vals = jnp.concatenate(
        [gwv, tg, ag, jnp.zeros((1, 122), jnp.float32)], axis=1)
    o_ref[...] = vals


def _proj_kernel(x_ref, w_ref, b_ref, o_ref):
    o_ref[...] = jnp.dot(x_ref[...], w_ref[...],
                         preferred_element_type=jnp.float32) + b_ref[...]


def _attn_kernel(q_ref, k_ref, v_ref, o_ref):
    for h in range(H):
        sl = slice(h * DH, (h + 1) * DH)
        s = jnp.dot(q_ref[:, sl], k_ref[:, sl].T,
                    preferred_element_type=jnp.float32) * SCALE
        m = jnp.max(s, axis=-1, keepdims=True)
        e = jnp.exp(s - m)
        o = jnp.dot(e, v_ref[:, sl], preferred_element_type=jnp.float32)
        o_ref[:, sl] = o / jnp.sum(e, axis=-1, keepdims=True)


def _sparse_kernel(q_ref, k_ref, v_ref, o_ref):
    for h in range(H):
        sl = slice(h * DH, (h + 1) * DH)
        s = jnp.dot(q_ref[:, sl], k_ref[:, sl].T,
                    preferred_element_type=jnp.float32) * SCALE
        cur = s
        rowmax = None
        for i in range(KS - 1):
            m = jnp.max(cur, axis=-1, keepdims=True)
            if i == 0:
                rowmax = m
            cur = jnp.where(cur >= m, NEG, cur)
        thr = jnp.max(cur, axis=-1, keepdims=True)
        e = jnp.where(s >= thr, jnp.exp(s - rowmax), 0.0)
        o = jnp.dot(e, v_ref[:, sl], preferred_element_type=jnp.float32)
        o_ref[:, sl] = o / jnp.sum(e, axis=-1, keepdims=True)


def _perf_kernel(q_ref, k_ref, v_ref, f_ref, o_ref):
    def elu1(t):
        return jnp.where(t > 0, t + 1.0, jnp.exp(jnp.minimum(t, 0.0)))

    for h in range(H):
        sl = slice(h * DH, (h + 1) * DH)
        f = f_ref[h]
        qf = elu1(jnp.dot(q_ref[:, sl], f,
                          preferred_element_type=jnp.float32))
        kf = elu1(jnp.dot(k_ref[:, sl], f,
                          preferred_element_type=jnp.float32))
        v = v_ref[:, sl]
        kv = jax.lax.dot_general(kf, v, (((0,), (0,)), ((), ())),
                                 preferred_element_type=jnp.float32)
        ksum = jnp.sum(kf, axis=0, keepdims=True)
        z = 1.0 / (jnp.sum(qf * ksum, axis=1, keepdims=True) + 1e-6)
        o_ref[:, sl] = jnp.dot(qf, kv,
                               preferred_element_type=jnp.float32) * z


def _expert_kernel(ctx_ref, l_ref, wo_ref, bo_ref, o_ref):
    l = l_ref[:, :E]
    m1 = jnp.max(l, axis=-1, keepdims=True)
    l2 = jnp.where(l >= m1, NEG, l)
    m2 = jnp.max(l2, axis=-1, keepdims=True)
    e2 = jnp.exp(m2 - m1)
    den = 1.0 + e2
    g1 = 1.0 / den
    g2 = e2 / den
    w = jnp.where(l >= m1, g1, jnp.where(l >= m2, g2, 0.0))
    ctx = ctx_ref[...]
    total = jnp.zeros(o_ref.shape, jnp.float32)
    for e in range(E):
        total += w[:, e:e + 1] * (
            jnp.dot(ctx, wo_ref[e], preferred_element_type=jnp.float32)
            + bo_ref[e])
    o_ref[...] = total


def _taa_ln1_kernel(g_ref, sp_ref, pf_ref, wsp_ref, wpf_ref,
                    x_ref, moe_ref, lg_ref, lb_ref, o_ref):
    g0 = g_ref[0:1, 2:3]
    g1 = g_ref[0:1, 3:4]
    taa = (jnp.dot(sp_ref[...] * g0, wsp_ref[...],
                   preferred_element_type=jnp.float32)
           + jnp.dot(pf_ref[...] * g1, wpf_ref[...],
                     preferred_element_type=jnp.float32))
    aw = g_ref[0:1, 0:1]
    ag0 = g_ref[0:1, 4:5]
    ag1 = g_ref[0:1, 5:6]
    att = aw * (ag0 * moe_ref[...] + ag1 * taa) * DEPTH
    y = x_ref[...] + att
    m = jnp.mean(y, axis=-1, keepdims=True)
    v = jnp.mean((y - m) ** 2, axis=-1, keepdims=True)
    o_ref[...] = (y - m) / jnp.sqrt(v + 1e-5) * lg_ref[...] + lb_ref[...]


def _ffn_kernel(g_ref, y_ref, w1_ref, b1_ref, w2_ref, b2_ref,
                lg_ref, lb_ref, o_ref):
    fw = g_ref[0:1, 1:2]
    y = y_ref[...]
    h = jax.nn.gelu(jnp.dot(y, w1_ref[...],
                            preferred_element_type=jnp.float32) + b1_ref[...])
    z = jnp.dot(h, w2_ref[...], preferred_element_type=jnp.float32) + b2_ref[...]
    t = y + fw * z * DEPTH
    m = jnp.mean(t, axis=-1, keepdims=True)
    v = jnp.mean((t - m) ** 2, axis=-1, keepdims=True)
    o_ref[...] = (t - m) / jnp.sqrt(v + 1e-5) * lg_ref[...] + lb_ref[...]


def kernel(x, arg_w1, arg_b1, arg_w2, arg_b2, moe_wq, moe_wk, moe_wv,
           moe_router_w, moe_router_b, moe_wo, moe_bo, sp_wq, sp_wk, sp_wv,
           sp_wo, pf_wq, pf_wk, pf_wv, pf_wo, pf_feat, taa_gate_w, taa_gate_b,
           attn_gate_w, attn_gate_b, ffn_w1, ffn_b1, ffn_w2, ffn_b2,
           ln1_g, ln1_b, ln2_g, ln2_b):
    xs = x.reshape(S, D)

    gates = pl.pallas_call(
        _gates_kernel,
        out_shape=jax.ShapeDtypeStruct((1, 128), jnp.float32),
    )(xs, arg_w1, arg_b1.reshape(1, 128), arg_w2, arg_b2.reshape(1, 2),
      taa_gate_w, taa_gate_b.reshape(1, 2), attn_gate_w,
      attn_gate_b.reshape(1, 2))

    wcat = jnp.concatenate(
        [moe_wq, moe_wk, moe_wv, sp_wq, sp_wk, sp_wv, pf_wq, pf_wk, pf_wv,
         jnp.pad(moe_router_w, ((0, 0), (0, 128 - E)))], axis=1)
    bcat = jnp.concatenate(
        [jnp.zeros((9 * D,), jnp.float32),
         jnp.pad(moe_router_b, (0, 128 - E))]).reshape(1, NPROJ)
    proj = pl.pallas_call(
        _proj_kernel,
        grid=(NPROJ // NB,),
        in_specs=[
            pl.BlockSpec((S, D), lambda j: (0, 0)),
            pl.BlockSpec((D, NB), lambda j: (0, j)),
            pl.BlockSpec((1, NB), lambda j: (0, j)),
        ],
        out_specs=pl.BlockSpec((S, NB), lambda j: (0, j)),
        out_shape=jax.ShapeDtypeStruct((S, NPROJ), jnp.float32),
    )(xs, wcat, bcat)

    def attn(fn, qb, kb, vb):
        return pl.pallas_call(
            fn,
            grid=(S // BA,),
            in_specs=[
                pl.BlockSpec((BA, D), lambda i: (i, qb)),
                pl.BlockSpec((S, D), lambda i: (0, kb)),
                pl.BlockSpec((S, D), lambda i: (0, vb)),
            ],
            out_specs=pl.BlockSpec((BA, D), lambda i: (i, 0)),
            out_shape=jax.ShapeDtypeStruct((S, D), jnp.float32),
        )(proj, proj, proj)

    moe_ctx = attn(_attn_kernel, 0, 1, 2)
    sp_ctx = attn(_sparse_kernel, 3, 4, 5)

    pf_ctx = pl.pallas_call(
        _perf_kernel,
        grid=(1,),
        in_specs=[
            pl.BlockSpec((S, D), lambda i: (0, 6)),
            pl.BlockSpec((S, D), lambda i: (0, 7)),
            pl.BlockSpec((S, D), lambda i: (0, 8)),
            pl.BlockSpec((H, DH, KD), lambda i: (0, 0, 0)),
        ],
        out_specs=pl.BlockSpec((S, D), lambda i: (0, 0)),
        out_shape=jax.ShapeDtypeStruct((S, D), jnp.float32),
    )(proj, proj, proj, pf_feat)

    moe_out = pl.pallas_call(
        _expert_kernel,
        grid=(S // BS,),
        in_specs=[
            pl.BlockSpec((BS, D), lambda i: (i, 0)),
            pl.BlockSpec((BS, 128), lambda i: (i, 54)),
            pl.BlockSpec((E, D, D), lambda i: (0, 0, 0)),
            pl.BlockSpec((E, 1, D), lambda i: (0, 0, 0)),
        ],
        out_specs=pl.BlockSpec((BS, D), lambda i: (i, 0)),
        out_shape=jax.ShapeDtypeStruct((S, D), jnp.float32),
    )(moe_ctx, proj, moe_wo, moe_bo.reshape(E, 1, D))

    y = pl.pallas_call(
        _taa_ln1_kernel,
        grid=(S // BS,),
        in_specs=[
            pl.BlockSpec((1, 128), lambda i: (0, 0)),
            pl.BlockSpec((BS, D), lambda i: (i, 0)),
            pl.BlockSpec((BS, D), lambda i: (i, 0)),
            pl.BlockSpec((D, D), lambda i: (0, 0)),
            pl.BlockSpec((D, D), lambda i: (0, 0)),
            pl.BlockSpec((BS, D), lambda i: (i, 0)),
            pl.BlockSpec((BS, D), lambda i: (i, 0)),
            pl.BlockSpec((1, D), lambda i: (0, 0)),
            pl.BlockSpec((1, D), lambda i: (0, 0)),
        ],
        out_specs=pl.BlockSpec((BS, D), lambda i: (i, 0)),
        out_shape=jax.ShapeDtypeStruct((S, D), jnp.float32),
    )(gates, sp_ctx, pf_ctx, sp_wo, pf_wo, xs, moe_out,
      ln1_g.reshape(1, D), ln1_b.reshape(1, D))

    out = pl.pallas_call(
        _ffn_kernel,
        grid=(S // BS,),
        in_specs=[
            pl.BlockSpec((1, 128), lambda i: (0, 0)),
            pl.BlockSpec((BS, D), lambda i: (i, 0)),
            pl.BlockSpec((D, FF), lambda i: (0, 0)),
            pl.BlockSpec((1, FF), lambda i: (0, 0)),
            pl.BlockSpec((FF, D), lambda i: (0, 0)),
            pl.BlockSpec((1, D), lambda i: (0, 0)),
            pl.BlockSpec((1, D), lambda i: (0, 0)),
            pl.BlockSpec((1, D), lambda i: (0, 0)),
        ],
        out_specs=pl.BlockSpec((BS, D), lambda i: (i, 0)),
        out_shape=jax.ShapeDtypeStruct((S, D), jnp.float32),
    )(gates, y, ffn_w1, ffn_b1.reshape(1, FF), ffn_w2,
      ffn_b2.reshape(1, D), ln2_g.reshape(1, D), ln2_b.reshape(1, D))

    return out.reshape(1, S, D)


# speedup vs baseline: 1.1011x; 1.0047x over previous
import numpy as np
import jax
import jax.numpy as jnp
from jax.experimental import pallas as pl

D = 768
H = 12
DH = 64
E = 8
FF = 3072
KD = 256
KS = 10
S = 2048
DEPTH = 1.25
SCALE = 1.0 / np.sqrt(DH)
BS = 512
BA = 256
NPROJ = 9 * D + 128
NB = 1408
NEG = -1e30


def _f32(*dots):
    return dict(preferred_element_type=jnp.float32)


def _gates_kernel(x_ref, aw1_ref, ab1_ref, aw2_ref, ab2_ref,
                  tw_ref, tb_ref, gw_ref, gb_ref, o_ref):
    avg = jnp.mean(x_ref[...], axis=0, keepdims=True)
    h = jax.nn.gelu(jnp.dot(avg, aw1_ref[...],
                            preferred_element_type=jnp.float32) + ab1_ref[...])
    gwv = jax.nn.sigmoid(jnp.dot(h, aw2_ref[...],
                                 preferred_element_type=jnp.float32) + ab2_ref[...])
    tg = jax.nn.softmax(jnp.dot(avg, tw_ref[...],
                                preferred_element_type=jnp.float32) + tb_ref[...], -1)
    ag = jax.nn.softmax(jnp.dot(avg, gw_ref[...],
                                preferred_element_type=jnp.float32) + gb_ref[...], -1)
    vals = jnp.concatenate(
        [gwv, tg, ag, jnp.zeros((1, 122), jnp.float32)], axis=1)
    o_ref[...] = vals


def _proj_kernel(x_ref, w_ref, b_ref, o_ref):
    o_ref[...] = jnp.dot(x_ref[...], w_ref[...],
                         preferred_element_type=jnp.float32) + b_ref[...]


def _attn_expert_kernel(q_ref, k_ref, v_ref, l_ref, wo_ref, bo_ref, o_ref):
    cols = []
    for h in range(H):
        sl = slice(h * DH, (h + 1) * DH)
        s = jnp.dot(q_ref[:, sl], k_ref[:, sl].T,
                    preferred_element_type=jnp.float32) * SCALE
        m = jnp.max(s, axis=-1, keepdims=True)
        e = jnp.exp(s - m)
        o = jnp.dot(e, v_ref[:, sl], preferred_element_type=jnp.float32)
        cols.append(o / jnp.sum(e, axis=-1, keepdims=True))
    ctx = jnp.concatenate(cols, axis=1)
    l = l_ref[:, :E]
    m1 = jnp.max(l, axis=-1, keepdims=True)
    l2 = jnp.where(l >= m1, NEG, l)
    m2 = jnp.max(l2, axis=-1, keepdims=True)
    e2 = jnp.exp(m2 - m1)
    den = 1.0 + e2
    w = jnp.where(l >= m1, 1.0 / den, jnp.where(l >= m2, e2 / den, 0.0))
    total = jnp.zeros(o_ref.shape, jnp.float32)
    for e in range(E):
        total += w[:, e:e + 1] * (
            jnp.dot(ctx, wo_ref[e], preferred_element_type=jnp.float32)
            + bo_ref[e])
    o_ref[...] = total


def _sparse_kernel(q_ref, k_ref, v_ref, o_ref):
    for h in range(H):
        sl = slice(h * DH, (h + 1) * DH)
        s = jnp.dot(q_ref[:, sl], k_ref[:, sl].T,
                    preferred_element_type=jnp.float32) * SCALE
        cur = s
        rowmax = None
        for i in range(KS - 1):
            m = jnp.max(cur, axis=-1, keepdims=True)
            if i == 0:
                rowmax = m
            cur = jnp.where(cur >= m, NEG, cur)
        thr = jnp.max(cur, axis=-1, keepdims=True)
        e = jnp.where(s >= thr, jnp.exp(s - rowmax), 0.0)
        o = jnp.dot(e, v_ref[:, sl], preferred_element_type=jnp.float32)
        o_ref[:, sl] = o / jnp.sum(e, axis=-1, keepdims=True)


def _perf_kernel(q_ref, k_ref, v_ref, f_ref, o_ref):
    def elu1(t):
        return jnp.where(t > 0, t + 1.0, jnp.exp(jnp.minimum(t, 0.0)))

    for h in range(H):
        sl = slice(h * DH, (h + 1) * DH)
        f = f_ref[h]
        qf = elu1(jnp.dot(q_ref[:, sl], f,
                          preferred_element_type=jnp.float32))
        kf = elu1(jnp.dot(k_ref[:, sl], f,
                          preferred_element_type=jnp.float32))
        v = v_ref[:, sl]
        kv = jax.lax.dot_general(kf, v, (((0,), (0,)), ((), ())),
                                 preferred_element_type=jnp.float32)
        ksum = jnp.sum(kf, axis=0, keepdims=True)
        z = 1.0 / (jnp.sum(qf * ksum, axis=1, keepdims=True) + 1e-6)
        o_ref[:, sl] = jnp.dot(qf, kv,
                               preferred_element_type=jnp.float32) * z


def _taa_ln1_kernel(g_ref, sp_ref, pf_ref, wsp_ref, wpf_ref,
                    x_ref, moe_ref, lg_ref, lb_ref, o_ref):
    g0 = g_ref[0:1, 2:3]
    g1 = g_ref[0:1, 3:4]
    taa = (jnp.dot(sp_ref[...] * g0, wsp_ref[...],
                   preferred_element_type=jnp.float32)
           + jnp.dot(pf_ref[...] * g1, wpf_ref[...],
                     preferred_element_type=jnp.float32))
    aw = g_ref[0:1, 0:1]
    ag0 = g_ref[0:1, 4:5]
    ag1 = g_ref[0:1, 5:6]
    att = aw * (ag0 * moe_ref[...] + ag1 * taa) * DEPTH
    y = x_ref[...] + att
    m = jnp.mean(y, axis=-1, keepdims=True)
    v = jnp.mean((y - m) ** 2, axis=-1, keepdims=True)
    o_ref[...] = (y - m) / jnp.sqrt(v + 1e-5) * lg_ref[...] + lb_ref[...]


def _ffn_kernel(g_ref, y_ref, w1_ref, b1_ref, w2_ref, b2_ref,
                lg_ref, lb_ref, o_ref):
    fw = g_ref[0:1, 1:2]
    y = y_ref[...]
    h = jax.nn.gelu(jnp.dot(y, w1_ref[...],
                            preferred_element_type=jnp.float32) + b1_ref[...])
    z = jnp.dot(h, w2_ref[...], preferred_element_type=jnp.float32) + b2_ref[...]
    t = y + fw * z * DEPTH
    m = jnp.mean(t, axis=-1, keepdims=True)
    v = jnp.mean((t - m) ** 2, axis=-1, keepdims=True)
    o_ref[...] = (t - m) / jnp.sqrt(v + 1e-5) * lg_ref[...] + lb_ref[...]


def kernel(x, arg_w1, arg_b1, arg_w2, arg_b2, moe_wq, moe_wk, moe_wv,
           moe_router_w, moe_router_b, moe_wo, moe_bo, sp_wq, sp_wk, sp_wv,
           sp_wo, pf_wq, pf_wk, pf_wv, pf_wo, pf_feat, taa_gate_w, taa_gate_b,
           attn_gate_w, attn_gate_b, ffn_w1, ffn_b1, ffn_w2, ffn_b2,
           ln1_g, ln1_b, ln2_g, ln2_b):
    xs = x.reshape(S, D)

    gates = pl.pallas_call(
        _gates_kernel,
        out_shape=jax.ShapeDtypeStruct((1, 128), jnp.float32),
    )(xs, arg_w1, arg_b1.reshape(1, 128), arg_w2, arg_b2.reshape(1, 2),
      taa_gate_w, taa_gate_b.reshape(1, 2), attn_gate_w,
      attn_gate_b.reshape(1, 2))

    wcat = jnp.concatenate(
        [moe_wq, moe_wk, moe_wv, sp_wq, sp_wk, sp_wv, pf_wq, pf_wk, pf_wv,
         jnp.pad(moe_router_w, ((0, 0), (0, 128 - E)))], axis=1)
    bcat = jnp.concatenate(
        [jnp.zeros((9 * D,), jnp.float32),
         jnp.pad(moe_router_b, (0, 128 - E))]).reshape(1, NPROJ)
    proj = pl.pallas_call(
        _proj_kernel,
        grid=(NPROJ // NB,),
        in_specs=[
            pl.BlockSpec((S, D), lambda j: (0, 0)),
            pl.BlockSpec((D, NB), lambda j: (0, j)),
            pl.BlockSpec((1, NB), lambda j: (0, j)),
        ],
        out_specs=pl.BlockSpec((S, NB), lambda j: (0, j)),
        out_shape=jax.ShapeDtypeStruct((S, NPROJ), jnp.float32),
    )(xs, wcat, bcat)

    def attn(fn, qb, kb, vb):
        return pl.pallas_call(
            fn,
            grid=(S // BA,),
            in_specs=[
                pl.BlockSpec((BA, D), lambda i: (i, qb)),
                pl.BlockSpec((S, D), lambda i: (0, kb)),
                pl.BlockSpec((S, D), lambda i: (0, vb)),
            ],
            out_specs=pl.BlockSpec((BA, D), lambda i: (i, 0)),
            out_shape=jax.ShapeDtypeStruct((S, D), jnp.float32),
        )(proj, proj, proj)

    sp_ctx = attn(_sparse_kernel, 3, 4, 5)

    moe_out = pl.pallas_call(
        _attn_expert_kernel,
        grid=(S // BA,),
        in_specs=[
            pl.BlockSpec((BA, D), lambda i: (i, 0)),
            pl.BlockSpec((S, D), lambda i: (0, 1)),
            pl.BlockSpec((S, D), lambda i: (0, 2)),
            pl.BlockSpec((BA, 128), lambda i: (i, 54)),
            pl.BlockSpec((E, D, D), lambda i: (0, 0, 0)),
            pl.BlockSpec((E, 1, D), lambda i: (0, 0, 0)),
        ],
        out_specs=pl.BlockSpec((BA, D), lambda i: (i, 0)),
        out_shape=jax.ShapeDtypeStruct((S, D), jnp.float32),
    )(proj, proj, proj, proj, moe_wo, moe_bo.reshape(E, 1, D))

    pf_ctx = pl.pallas_call(
        _perf_kernel,
        grid=(1,),
        in_specs=[
            pl.BlockSpec((S, D), lambda i: (0, 6)),
            pl.BlockSpec((S, D), lambda i: (0, 7)),
            pl.BlockSpec((S, D), lambda i: (0, 8)),
            pl.BlockSpec((H, DH, KD), lambda i: (0, 0, 0)),
        ],
        out_specs=pl.BlockSpec((S, D), lambda i: (0, 0)),
        out_shape=jax.ShapeDtypeStruct((S, D), jnp.float32),
    )(proj, proj, proj, pf_feat)

    y = pl.pallas_call(
        _taa_ln1_kernel,
        grid=(S // BS,),
        in_specs=[
            pl.BlockSpec((1, 128), lambda i: (0, 0)),
            pl.BlockSpec((BS, D), lambda i: (i, 0)),
            pl.BlockSpec((BS, D), lambda i: (i, 0)),
            pl.BlockSpec((D, D), lambda i: (0, 0)),
            pl.BlockSpec((D, D), lambda i: (0, 0)),
            pl.BlockSpec((BS, D), lambda i: (i, 0)),
            pl.BlockSpec((BS, D), lambda i: (i, 0)),
            pl.BlockSpec((1, D), lambda i: (0, 0)),
            pl.BlockSpec((1, D), lambda i: (0, 0)),
        ],
        out_specs=pl.BlockSpec((BS, D), lambda i: (i, 0)),
        out_shape=jax.ShapeDtypeStruct((S, D), jnp.float32),
    )(gates, sp_ctx, pf_ctx, sp_wo, pf_wo, xs, moe_out,
      ln1_g.reshape(1, D), ln1_b.reshape(1, D))

    out = pl.pallas_call(
        _ffn_kernel,
        grid=(S // BS,),
        in_specs=[
            pl.BlockSpec((1, 128), lambda i: (0, 0)),
            pl.BlockSpec((BS, D), lambda i: (i, 0)),
            pl.BlockSpec((D, FF), lambda i: (0, 0)),
            pl.BlockSpec((1, FF), lambda i: (0, 0)),
            pl.BlockSpec((FF, D), lambda i: (0, 0)),
            pl.BlockSpec((1, D), lambda i: (0, 0)),
            pl.BlockSpec((1, D), lambda i: (0, 0)),
            pl.BlockSpec((1, D), lambda i: (0, 0)),
        ],
        out_specs=pl.BlockSpec((BS, D), lambda i: (i, 0)),
        out_shape=jax.ShapeDtypeStruct((S, D), jnp.float32),
    )(gates, y, ffn_w1, ffn_b1.reshape(1, FF), ffn_w2,
      ffn_b2.reshape(1, D), ln2_g.reshape(1, D), ln2_b.reshape(1, D))

    return out.reshape(1, S, D)
